# initial kernel scaffold (unmeasured)
import jax
import jax.numpy as jnp
from jax import lax
from jax.experimental import pallas as pl
from jax.experimental.pallas import tpu as pltpu

N_DEV = 16
M_BLK = 512
K_BLK = 512


def kernel(x, w_mat):
    m_total, k_shard = x.shape
    k_total, n = w_mat.shape
    assert k_shard == K_BLK and m_total == N_DEV * M_BLK

    def body(x_ref, w_ref, out_ref, a2a_buf, w_buf,
             send_sems, recv_sems, w_sems, loc_sem):
        me = lax.axis_index("i")

        rdmas = []
        for d in range(1, N_DEV):
            t = lax.rem(me + d, N_DEV)
            rdma = pltpu.make_async_remote_copy(
                src_ref=x_ref.at[pl.ds(t * M_BLK, M_BLK), :],
                dst_ref=a2a_buf.at[d],
                send_sem=send_sems.at[d],
                recv_sem=recv_sems.at[d],
                device_id=(t,),
                device_id_type=pl.DeviceIdType.MESH,
            )
            rdma.start()
            rdmas.append(rdma)

        loc = pltpu.make_async_copy(
            x_ref.at[pl.ds(me * M_BLK, M_BLK), :], a2a_buf.at[0], loc_sem)
        loc.start()

        def w_copy(d, slot):
            s = lax.rem(me + (N_DEV - d), N_DEV)
            return pltpu.make_async_copy(
                w_ref.at[pl.ds(s * K_BLK, K_BLK), :],
                w_buf.at[slot], w_sems.at[slot])

        w_copy(0, 0).start()

        for d in range(N_DEV):
            if d + 1 < N_DEV:
                w_copy(d + 1, (d + 1) % 2).start()
            w_copy(d, d % 2).wait()
            if d == 0:
                loc.wait()
            else:
                rdmas[d - 1].wait_recv()
            prod = lax.dot_general(
                a2a_buf[d], w_buf[d % 2],
                (((1,), (0,)), ((), ())),
                preferred_element_type=jnp.float32,
            )
            if d == 0:
                out_ref[...] = prod
            else:
                out_ref[...] += prod

        out_ref[...] = jnp.maximum(out_ref[...], 0.0)

        for r in rdmas:
            r.wait_send()

    return pl.pallas_call(
        body,
        out_shape=jax.ShapeDtypeStruct((M_BLK, n), jnp.float32),
        in_specs=[
            pl.BlockSpec(memory_space=pltpu.ANY),
            pl.BlockSpec(memory_space=pltpu.ANY),
        ],
        out_specs=pl.BlockSpec(memory_space=pltpu.VMEM),
        scratch_shapes=[
            pltpu.VMEM((N_DEV, M_BLK, K_BLK), jnp.float32),
            pltpu.VMEM((2, K_BLK, n), jnp.float32),
            pltpu.SemaphoreType.DMA((N_DEV,)),
            pltpu.SemaphoreType.DMA((N_DEV,)),
            pltpu.SemaphoreType.DMA((2,)),
            pltpu.SemaphoreType.DMA,
        ],
    )(x, w_mat)


# baseline (device time: 198293 ns/iter reference)
import jax
import jax.numpy as jnp
from jax import lax
from jax.experimental import pallas as pl
from jax.experimental.pallas import tpu as pltpu

N_DEV = 16
M_BLK = 512
K_BLK = 512


def kernel(x, w_mat):
    m_total, k_shard = x.shape
    k_total, n = w_mat.shape
    assert k_shard == K_BLK and m_total == N_DEV * M_BLK

    def body(x_ref, w_ref, out_ref, a2a_buf, w_buf,
             send_sems, recv_sems, w_sems, loc_sem):
        me = lax.axis_index("i")

        barrier_sem = pltpu.get_barrier_semaphore()
        for d in range(1, N_DEV):
            t = lax.rem(me + d, N_DEV)
            pl.semaphore_signal(
                barrier_sem, inc=1,
                device_id=(t,), device_id_type=pl.DeviceIdType.MESH)
        pl.semaphore_wait(barrier_sem, N_DEV - 1)

        rdmas = []
        for d in range(1, N_DEV):
            t = lax.rem(me + d, N_DEV)
            rdma = pltpu.make_async_remote_copy(
                src_ref=x_ref.at[pl.ds(t * M_BLK, M_BLK), :],
                dst_ref=a2a_buf.at[d],
                send_sem=send_sems.at[d],
                recv_sem=recv_sems.at[d],
                device_id=(t,),
                device_id_type=pl.DeviceIdType.MESH,
            )
            rdma.start()
            rdmas.append(rdma)

        loc = pltpu.make_async_copy(
            x_ref.at[pl.ds(me * M_BLK, M_BLK), :], a2a_buf.at[0], loc_sem)
        loc.start()

        def w_copy(d, slot):
            s = lax.rem(me + (N_DEV - d), N_DEV)
            return pltpu.make_async_copy(
                w_ref.at[pl.ds(s * K_BLK, K_BLK), :],
                w_buf.at[slot], w_sems.at[slot])

        w_copy(0, 0).start()

        for d in range(N_DEV):
            if d + 1 < N_DEV:
                w_copy(d + 1, (d + 1) % 2).start()
            w_copy(d, d % 2).wait()
            if d == 0:
                loc.wait()
            else:
                rdmas[d - 1].wait_recv()
            prod = lax.dot_general(
                a2a_buf[d], w_buf[d % 2],
                (((1,), (0,)), ((), ())),
                preferred_element_type=jnp.float32,
            )
            if d == 0:
                out_ref[...] = prod
            else:
                out_ref[...] += prod

        out_ref[...] = jnp.maximum(out_ref[...], 0.0)

        for r in rdmas:
            r.wait_send()

    return pl.pallas_call(
        body,
        out_shape=jax.ShapeDtypeStruct((M_BLK, n), jnp.float32),
        in_specs=[
            pl.BlockSpec(memory_space=pl.ANY),
            pl.BlockSpec(memory_space=pl.ANY),
        ],
        out_specs=pl.BlockSpec(memory_space=pltpu.VMEM),
        scratch_shapes=[
            pltpu.VMEM((N_DEV, M_BLK, K_BLK), jnp.float32),
            pltpu.VMEM((2, K_BLK, n), jnp.float32),
            pltpu.SemaphoreType.DMA((N_DEV,)),
            pltpu.SemaphoreType.DMA((N_DEV,)),
            pltpu.SemaphoreType.DMA((2,)),
            pltpu.SemaphoreType.DMA,
        ],
        compiler_params=pltpu.CompilerParams(collective_id=0),
    )(x, w_mat)


# device time: 115024 ns/iter; 1.7239x vs baseline; 1.7239x over previous
import jax
import jax.numpy as jnp
from jax import lax
from jax.experimental import pallas as pl
from jax.experimental.pallas import tpu as pltpu

N_DEV = 16
M_BLK = 512
K_BLK = 512
N_HALF = 2048


def kernel(x, w_mat):
    m_total, k_shard = x.shape
    k_total, n = w_mat.shape
    assert k_shard == K_BLK and m_total == N_DEV * M_BLK
    n_halves = n // N_HALF
    n_steps = N_DEV * n_halves

    def body(x_ref, w_ref, out_ref, staging, x_send, a2a_buf, w_buf,
             send_sems, recv_sems, st_sems, w_sems):
        me = lax.axis_index("i")

        barrier_sem = pltpu.get_barrier_semaphore()
        for d in range(1, N_DEV):
            t = lax.rem(me + d, N_DEV)
            pl.semaphore_signal(
                barrier_sem, inc=1,
                device_id=(t,), device_id_type=pl.DeviceIdType.MESH)
        pl.semaphore_wait(barrier_sem, N_DEV - 1)

        def stage_in(j, slot):
            t = lax.rem(me + j, N_DEV)
            return pltpu.make_async_copy(
                x_ref.at[pl.ds(t * M_BLK, M_BLK), :],
                staging.at[slot], st_sems.at[slot])

        def w_copy(step, slot):
            d, h = divmod(step, n_halves)
            s = lax.rem(me + (N_DEV - d), N_DEV)
            return pltpu.make_async_copy(
                w_ref.at[pl.ds(s * K_BLK, K_BLK), pl.ds(h * N_HALF, N_HALF)],
                w_buf.at[slot], w_sems.at[slot])

        w_copy(0, 0).start()

        seq = list(range(1, N_DEV)) + [0]
        stage_in(seq[0], 0).start()
        rdmas = []
        for idx, j in enumerate(seq):
            slot = idx % 2
            if idx + 1 < len(seq):
                stage_in(seq[idx + 1], (idx + 1) % 2).start()
            stage_in(j, slot).wait()
            x_send[j] = staging[slot].astype(jnp.bfloat16)
            if j != 0:
                t = lax.rem(me + j, N_DEV)
                rdma = pltpu.make_async_remote_copy(
                    src_ref=x_send.at[j],
                    dst_ref=a2a_buf.at[j],
                    send_sem=send_sems.at[j],
                    recv_sem=recv_sems.at[j],
                    device_id=(t,),
                    device_id_type=pl.DeviceIdType.MESH,
                )
                rdma.start()
                rdmas.append(rdma)

        for d in range(N_DEV):
            for h in range(n_halves):
                step = d * n_halves + h
                if step + 1 < n_steps:
                    w_copy(step + 1, (step + 1) % 2).start()
                w_copy(step, step % 2).wait()
                if h == 0 and d > 0:
                    rdmas[d - 1].wait_recv()
                lhs = (x_send[0] if d == 0 else a2a_buf[d]).astype(jnp.float32)
                prod = lax.dot_general(
                    lhs, w_buf[step % 2],
                    (((1,), (0,)), ((), ())),
                    preferred_element_type=jnp.float32,
                )
                cols = pl.ds(h * N_HALF, N_HALF)
                if d == 0:
                    out_ref[:, cols] = prod
                else:
                    out_ref[:, cols] += prod

        out_ref[...] = jnp.maximum(out_ref[...], 0.0)

        for r in rdmas:
            r.wait_send()

    return pl.pallas_call(
        body,
        out_shape=jax.ShapeDtypeStruct((M_BLK, n), jnp.float32),
        in_specs=[
            pl.BlockSpec(memory_space=pl.ANY),
            pl.BlockSpec(memory_space=pl.ANY),
        ],
        out_specs=pl.BlockSpec(memory_space=pltpu.MemorySpace.VMEM),
        scratch_shapes=[
            pltpu.VMEM((2, M_BLK, K_BLK), jnp.float32),
            pltpu.VMEM((N_DEV, M_BLK, K_BLK), jnp.bfloat16),
            pltpu.VMEM((N_DEV, M_BLK, K_BLK), jnp.bfloat16),
            pltpu.VMEM((2, K_BLK, N_HALF), jnp.float32),
            pltpu.SemaphoreType.DMA((N_DEV,)),
            pltpu.SemaphoreType.DMA((N_DEV,)),
            pltpu.SemaphoreType.DMA((2,)),
            pltpu.SemaphoreType.DMA((2,)),
        ],
        compiler_params=pltpu.CompilerParams(collective_id=0),
    )(x, w_mat)
